# baseline (speedup 1.0000x reference)
import functools

import jax
import jax.numpy as jnp
from jax import lax
from jax.experimental import pallas as pl
from jax.experimental.pallas import tpu as pltpu
from jax.experimental.pallas import tpu_sc as plsc

_D = 64
_EPS = 1e-12
_CB = 8


def _sc_gather(token_table, idx_a, idx_b, b, s):
    n_rows = b * s
    sa = idx_a.shape[1]
    sb = s - sa
    info = plsc.get_sparse_core_info()
    nw = info.num_cores * info.num_subcores
    bpw = b // nw
    n_chunks = bpw // _CB
    mesh = plsc.VectorSubcoreMesh(core_axis_name="c", subcore_axis_name="s")

    @functools.partial(
        pl.kernel,
        mesh=mesh,
        compiler_params=pltpu.CompilerParams(use_tc_tiling_on_sc=False),
        out_type=jax.ShapeDtypeStruct((n_rows, _D), jnp.float32),
        scratch_types=[
            pltpu.VMEM((_CB, sa), jnp.int32),
            pltpu.VMEM((_CB, sa), jnp.int32),
            pltpu.VMEM((_CB * s, _D), jnp.float32),
            pltpu.SemaphoreType.DMA,
        ],
    )
    def k(table_hbm, ia_hbm, ib_hbm, out_hbm, ia_v, ib_v, rows_v, sem):
        cid = lax.axis_index("c")
        sid = lax.axis_index("s")
        wid = sid * info.num_cores + cid

        def chunk(g, carry):
            r0 = wid * bpw + g * _CB
            pltpu.sync_copy(ia_hbm.at[pl.ds(r0, _CB)], ia_v)
            pltpu.sync_copy(ib_hbm.at[pl.ds(r0, _CB)], ib_v)
            copies = []
            for i in range(_CB):
                copies.append(
                    pltpu.async_copy(
                        table_hbm.at[ia_v.at[i]],
                        rows_v.at[pl.ds(i * s, sa)],
                        sem,
                    )
                )
                copies.append(
                    pltpu.async_copy(
                        table_hbm.at[ib_v.at[i, pl.ds(0, sb)]],
                        rows_v.at[pl.ds(i * s + sa, sb)],
                        sem,
                    )
                )
            for c in copies:
                c.wait()
            pltpu.sync_copy(rows_v, out_hbm.at[pl.ds(r0 * s, _CB * s)])
            return carry

        lax.fori_loop(0, n_chunks, chunk, 0)

    return k(token_table, idx_a, idx_b)


def _tc_layernorm(x2, pos2, gb2, b, s):
    rows = x2.shape[0]
    sp = s // 2
    bb = 4 * sp
    rep = bb // sp

    def body(x_ref, pos_ref, gb_ref, o_ref):
        x = x_ref[...] + jnp.tile(pos_ref[...], (rep, 1))
        xa = x[:, :_D]
        xb = x[:, _D:]

        def ln(h):
            m = jnp.mean(h, axis=-1, keepdims=True)
            c = h - m
            v = jnp.mean(c * c, axis=-1, keepdims=True)
            return c * lax.rsqrt(v + _EPS)

        y = jnp.concatenate([ln(xa), ln(xb)], axis=-1)
        o_ref[...] = y * gb_ref[0:1, :] + gb_ref[1:2, :]

    return pl.pallas_call(
        body,
        grid=(rows // bb,),
        in_specs=[
            pl.BlockSpec((bb, 2 * _D), lambda i: (i, 0)),
            pl.BlockSpec((sp, 2 * _D), lambda i: (0, 0)),
            pl.BlockSpec((2, 2 * _D), lambda i: (0, 0)),
        ],
        out_specs=pl.BlockSpec((bb, 2 * _D), lambda i: (i, 0)),
        out_shape=jax.ShapeDtypeStruct((rows, 2 * _D), jnp.float32),
    )(x2, pos2, gb2)


def kernel(input_ids, token_table, pos_table, gamma, beta):
    b, s = input_ids.shape
    idx_a = input_ids[:, :128]
    idx_b = jnp.pad(input_ids[:, 128:], ((0, 0), (0, 128 - (s - 128))))
    gathered = _sc_gather(token_table, idx_a, idx_b, b, s)
    x2 = gathered.reshape(b * s // 2, 2 * _D)
    pos2 = pos_table.reshape(s // 2, 2 * _D)
    gb2 = jnp.stack([jnp.concatenate([gamma, gamma]),
                     jnp.concatenate([beta, beta])])
    return _tc_layernorm(x2, pos2, gb2, b, s).reshape(b, s, _D)


# speedup vs baseline: 1.1074x; 1.1074x over previous
import functools

import jax
import jax.numpy as jnp
from jax import lax
from jax.experimental import pallas as pl
from jax.experimental.pallas import tpu as pltpu
from jax.experimental.pallas import tpu_sc as plsc

_D = 64
_EPS = 1e-12
_CB = 8


def _sc_gather(token_table, idx_a, idx_b, b, s):
    n_rows = b * s
    sa = idx_a.shape[1]
    sb = s - sa
    info = plsc.get_sparse_core_info()
    nw = info.num_cores * info.num_subcores
    bpw = b // nw
    n_chunks = bpw // _CB
    mesh = plsc.VectorSubcoreMesh(core_axis_name="c", subcore_axis_name="s")

    @functools.partial(
        pl.kernel,
        mesh=mesh,
        compiler_params=pltpu.CompilerParams(use_tc_tiling_on_sc=False),
        out_type=jax.ShapeDtypeStruct((n_rows, _D), jnp.float32),
        scratch_types=[
            pltpu.VMEM((_CB, sa), jnp.int32),
            pltpu.VMEM((_CB, sa), jnp.int32),
            pltpu.VMEM((_CB * s, _D), jnp.float32),
            pltpu.SemaphoreType.DMA,
        ],
    )
    def k(table_hbm, ia_hbm, ib_hbm, out_hbm, ia_v, ib_v, rows_v, sem):
        cid = lax.axis_index("c")
        sid = lax.axis_index("s")
        wid = sid * info.num_cores + cid

        def chunk(g, carry):
            r0 = wid * bpw + g * _CB
            pltpu.sync_copy(ia_hbm.at[pl.ds(r0, _CB)], ia_v)
            pltpu.sync_copy(ib_hbm.at[pl.ds(r0, _CB)], ib_v)
            copies = []
            for i in range(_CB):
                copies.append(
                    pltpu.async_copy(
                        table_hbm.at[ia_v.at[i]],
                        rows_v.at[pl.ds(i * s, sa)],
                        sem,
                    )
                )
                copies.append(
                    pltpu.async_copy(
                        table_hbm.at[ib_v.at[i, pl.ds(0, sb)]],
                        rows_v.at[pl.ds(i * s + sa, sb)],
                        sem,
                    )
                )
            for c in copies:
                c.wait()
            pltpu.sync_copy(rows_v, out_hbm.at[pl.ds(r0 * s, _CB * s)])
            return carry

        lax.fori_loop(0, n_chunks, chunk, 0)

    return k(token_table, idx_a, idx_b)


def _tc_layernorm(gathered3d, pos3d, gamma3d, beta3d):
    b, s, d = gathered3d.shape
    bb = 32

    def body(x_ref, pos_ref, gamma_ref, beta_ref, o_ref):
        x = x_ref[...] + pos_ref[...]
        mean = jnp.mean(x, axis=-1, keepdims=True)
        xc = x - mean
        var = jnp.mean(xc * xc, axis=-1, keepdims=True)
        o_ref[...] = (
            xc * lax.rsqrt(var + _EPS) * gamma_ref[...] + beta_ref[...]
        )

    return pl.pallas_call(
        body,
        grid=(b // bb,),
        in_specs=[
            pl.BlockSpec((bb, s, d), lambda i: (i, 0, 0)),
            pl.BlockSpec((1, s, d), lambda i: (0, 0, 0)),
            pl.BlockSpec((1, 1, d), lambda i: (0, 0, 0)),
            pl.BlockSpec((1, 1, d), lambda i: (0, 0, 0)),
        ],
        out_specs=pl.BlockSpec((bb, s, d), lambda i: (i, 0, 0)),
        out_shape=jax.ShapeDtypeStruct((b, s, d), jnp.float32),
    )(gathered3d, pos3d, gamma3d, beta3d)


def kernel(input_ids, token_table, pos_table, gamma, beta):
    b, s = input_ids.shape
    idx_a = input_ids[:, :128]
    idx_b = jnp.pad(input_ids[:, 128:], ((0, 0), (0, 128 - (s - 128))))
    gathered = _sc_gather(token_table + 0.0, idx_a, idx_b, b, s)
    return _tc_layernorm(
        gathered.reshape(b, s, _D),
        pos_table.reshape(1, s, _D),
        gamma.reshape(1, 1, _D),
        beta.reshape(1, 1, _D),
    )
